# baseline (speedup 1.0000x reference)
import functools

import jax
import jax.numpy as jnp
from jax import lax
from jax.experimental import pallas as pl
from jax.experimental.pallas import tpu as pltpu
from jax.experimental.pallas import tpu_sc as plsc

_NC = 2
_NS = 16
_NW = _NC * _NS
_L = 16

_CH = 128


def _transpose_tc(Y):
    K, NI = Y.shape
    TW = 512
    grid = (NI + TW - 1) // TW

    def body(y_ref, yt_ref):
        yt_ref[...] = y_ref[...].T

    return pl.pallas_call(
        body,
        grid=(grid,),
        in_specs=[pl.BlockSpec((K, TW), lambda i: (0, i))],
        out_specs=pl.BlockSpec((TW, K), lambda i: (i, 0)),
        out_shape=jax.ShapeDtypeStruct((NI, K), Y.dtype),
    )(Y)


def _sc_score(user_ids, pos_ids, neg_ids, X, YT):
    B = user_ids.shape[0]
    K = X.shape[1]
    assert K == 128
    bpw = B // _NW
    sc = 2 * _CH
    nsc = bpw // sc
    ngrp = sc // _L

    mesh = plsc.VectorSubcoreMesh(core_axis_name="c", subcore_axis_name="s")

    @functools.partial(
        pl.kernel,
        mesh=mesh,
        out_type=jax.ShapeDtypeStruct((B,), jnp.float32),
        scratch_types=[
            pltpu.VMEM((bpw,), jnp.int32),
            pltpu.VMEM((bpw,), jnp.int32),
            pltpu.VMEM((bpw,), jnp.int32),
            pltpu.VMEM((sc, 128), jnp.float32),
            pltpu.VMEM((sc, 128), jnp.float32),
            pltpu.VMEM((sc, 128), jnp.float32),
            pltpu.VMEM((bpw,), jnp.float32),
            pltpu.SemaphoreType.DMA,
        ],
    )
    def k(uid_hbm, pid_hbm, nid_hbm, x_hbm, yt_hbm, out_hbm,
          uix, pix, nix, ub, pb, nb, ob, sem):
        wid = lax.axis_index("s") * _NC + lax.axis_index("c")
        base = wid * bpw

        c1 = pltpu.async_copy(uid_hbm.at[pl.ds(base, bpw)], uix, sem)
        c2 = pltpu.async_copy(pid_hbm.at[pl.ds(base, bpw)], pix, sem)
        c3 = pltpu.async_copy(nid_hbm.at[pl.ds(base, bpw)], nix, sem)
        c1.wait()
        c2.wait()
        c3.wait()

        for s in range(nsc):
            waits = []
            for h in range(2):
                off = s * sc + h * _CH
                dst = pl.ds(h * _CH, _CH)
                waits.append(pltpu.async_copy(
                    x_hbm.at[uix.at[pl.ds(off, _CH)]], ub.at[dst], sem))
                waits.append(pltpu.async_copy(
                    yt_hbm.at[pix.at[pl.ds(off, _CH)]], pb.at[dst], sem))
                waits.append(pltpu.async_copy(
                    yt_hbm.at[nix.at[pl.ds(off, _CH)]], nb.at[dst], sem))
            for w in waits:
                w.wait()

            def grp(g, _, s=s):
                lanes = lax.iota(jnp.int32, _L)
                tot = jnp.zeros((_L,), jnp.float32)
                for e in range(_L):
                    r = g * _L + e
                    acc = jnp.zeros((_L,), jnp.float32)
                    for k in range(K // _L):
                        u = ub[r, pl.ds(k * _L, _L)]
                        p = pb[r, pl.ds(k * _L, _L)]
                        n = nb[r, pl.ds(k * _L, _L)]
                        acc = acc + u * (p - n)
                    for m in (8, 4, 2, 1):
                        acc = acc + acc.at[lanes ^ m].get(
                            mode="promise_in_bounds")
                    tot = jnp.where(lanes == e, acc, tot)
                ob[pl.ds(s * sc + g * _L, _L)] = tot
                return 0

            lax.fori_loop(0, ngrp, grp, 0)

        pltpu.sync_copy(ob, out_hbm.at[pl.ds(base, bpw)])

    return k(user_ids, pos_ids, neg_ids, X, YT)


def kernel(user_ids, pos_item_ids, neg_item_ids, X, Y):
    user_ids = user_ids.astype(jnp.int32)
    pos_item_ids = pos_item_ids.astype(jnp.int32)
    neg_item_ids = neg_item_ids.astype(jnp.int32)
    YT = _transpose_tc(Y)
    return _sc_score(user_ids, pos_item_ids, neg_item_ids, X, YT)


# speedup vs baseline: 3.4396x; 3.4396x over previous
import functools

import jax
import jax.numpy as jnp
from jax import lax
from jax.experimental import pallas as pl
from jax.experimental.pallas import tpu as pltpu
from jax.experimental.pallas import tpu_sc as plsc

_NC = 2
_NS = 16
_NW = _NC * _NS
_L = 16

_CH = 128


def _transpose_tc(Y):
    K, NI = Y.shape
    TW = 512
    grid = (NI + TW - 1) // TW

    def body(y_ref, yt_ref):
        yt_ref[...] = y_ref[...].T

    return pl.pallas_call(
        body,
        grid=(grid,),
        in_specs=[pl.BlockSpec((K, TW), lambda i: (0, i))],
        out_specs=pl.BlockSpec((TW, K), lambda i: (i, 0)),
        out_shape=jax.ShapeDtypeStruct((NI, K), Y.dtype),
    )(Y)


def _sc_score(user_ids, pos_ids, neg_ids, X, YT):
    B = user_ids.shape[0]
    K = X.shape[1]
    assert K == 128
    bpw = B // _NW
    sc = 2 * _CH
    nsc = bpw // sc
    ngrp = sc // _L

    mesh = plsc.VectorSubcoreMesh(core_axis_name="c", subcore_axis_name="s")

    @functools.partial(
        pl.kernel,
        mesh=mesh,
        out_type=jax.ShapeDtypeStruct((B,), jnp.float32),
        scratch_types=[
            pltpu.VMEM((bpw,), jnp.int32),
            pltpu.VMEM((bpw,), jnp.int32),
            pltpu.VMEM((bpw,), jnp.int32),
            pltpu.VMEM((sc, 128), jnp.float32),
            pltpu.VMEM((sc, 128), jnp.float32),
            pltpu.VMEM((sc, 128), jnp.float32),
            pltpu.VMEM((bpw,), jnp.float32),
            pltpu.SemaphoreType.DMA,
        ],
    )
    def k(uid_hbm, pid_hbm, nid_hbm, x_hbm, yt_hbm, out_hbm,
          uix, pix, nix, ub, pb, nb, ob, sem):
        wid = lax.axis_index("s") * _NC + lax.axis_index("c")
        base = wid * bpw

        c1 = pltpu.async_copy(uid_hbm.at[pl.ds(base, bpw)], uix, sem)
        c2 = pltpu.async_copy(pid_hbm.at[pl.ds(base, bpw)], pix, sem)
        c3 = pltpu.async_copy(nid_hbm.at[pl.ds(base, bpw)], nix, sem)
        c1.wait()
        c2.wait()
        c3.wait()

        for s in range(nsc):
            waits = []
            for h in range(2):
                off = s * sc + h * _CH
                dst = pl.ds(h * _CH, _CH)
                waits.append(pltpu.async_copy(
                    x_hbm.at[uix.at[pl.ds(off, _CH)]], ub.at[dst], sem))
                waits.append(pltpu.async_copy(
                    yt_hbm.at[pix.at[pl.ds(off, _CH)]], pb.at[dst], sem))
                waits.append(pltpu.async_copy(
                    yt_hbm.at[nix.at[pl.ds(off, _CH)]], nb.at[dst], sem))
            for w in waits:
                w.wait()

            def grp(g, _, s=s):
                lanes = lax.iota(jnp.int32, _L)
                tot = jnp.zeros((_L,), jnp.float32)
                for e in range(_L):
                    r = g * _L + e
                    acc = jnp.zeros((_L,), jnp.float32)
                    for k in range(K // _L):
                        u = ub[r, pl.ds(k * _L, _L)]
                        p = pb[r, pl.ds(k * _L, _L)]
                        n = nb[r, pl.ds(k * _L, _L)]
                        acc = acc + u * (p - n)
                    for m in (8, 4, 2, 1):
                        acc = acc + acc.at[lanes ^ m].get(
                            mode="promise_in_bounds")
                    tot = jnp.where(lanes == e, acc, tot)
                ob[pl.ds(s * sc + g * _L, _L)] = tot
                return 0

            lax.fori_loop(0, ngrp, grp, 0)

        pltpu.sync_copy(ob, out_hbm.at[pl.ds(base, bpw)])

    return k(user_ids, pos_ids, neg_ids, X, YT)


def kernel(user_ids, pos_item_ids, neg_item_ids, X, Y):
    user_ids = user_ids.astype(jnp.int32)
    pos_item_ids = pos_item_ids.astype(jnp.int32)
    neg_item_ids = neg_item_ids.astype(jnp.int32)
    YT = jnp.transpose(Y)
    return _sc_score(user_ids, pos_item_ids, neg_item_ids, X, YT)
